# baseline (speedup 1.0000x reference)
import functools

import jax
import jax.numpy as jnp
from jax import lax
from jax.experimental import pallas as pl
from jax.experimental.pallas import tpu as pltpu
from jax.experimental.pallas import tpu_sc as plsc

N = 10000
D = 128
NPAD = 10240
K = 320
LANES = 16


def _zero_vmem_1d(ref, n):
    def body(i, _):
        ref[pl.ds(i * LANES, LANES)] = jnp.zeros((LANES,), jnp.float32)
        return 0
    lax.fori_loop(0, n // LANES, body, 0)


def _zero_vmem_2d(ref, rows):
    def body(i, _):
        for k in range(D // LANES):
            ref[i, pl.ds(k * LANES, LANES)] = jnp.zeros((LANES,), jnp.float32)
        return 0
    lax.fori_loop(0, rows, body, 0)


KD = 2000


def _make_deg_kernel(E, nc, ns):
    mesh = plsc.VectorSubcoreMesh(core_axis_name="c", subcore_axis_name="s")
    nt = nc * ns
    T = E // (KD * nt)
    slot = NPAD // ns

    @functools.partial(
        pl.kernel,
        out_type=(
            jax.ShapeDtypeStruct((nc * NPAD,), jnp.float32),
            jax.ShapeDtypeStruct((nc * NPAD,), jnp.float32),
        ),
        mesh=mesh,
        scratch_types=[
            pltpu.VMEM_SHARED((NPAD,), jnp.float32),
            pltpu.VMEM_SHARED((NPAD,), jnp.float32),
            pltpu.VMEM((KD,), jnp.int32),
            pltpu.VMEM((KD,), jnp.int32),
            pltpu.VMEM((KD,), jnp.int32),
            pltpu.VMEM((KD,), jnp.int32),
            pltpu.VMEM((KD,), jnp.float32),
            pltpu.VMEM((slot,), jnp.float32),
            pltpu.SemaphoreType.DMA,
            pltpu.SemaphoreType.DMA,
            pltpu.SemaphoreType.DMA,
            pltpu.SemaphoreType.DMA,
        ],
    )
    def deg_kernel(eflat, odeg_hbm, ideg_hbm,
                   histS_sh, histD_sh,
                   idxS0, idxS1, idxD0, idxD1, ones_v, zbuf,
                   semS0, semS1, semD0, semD1):
        c = lax.axis_index("c")
        s = lax.axis_index("s")
        w = c * ns + s
        _zero_vmem_1d(zbuf, slot)
        for j in range(KD // LANES):
            ones_v[pl.ds(j * LANES, LANES)] = jnp.ones((LANES,), jnp.float32)
        pltpu.sync_copy(zbuf, histS_sh.at[pl.ds(s * slot, slot)])
        pltpu.sync_copy(zbuf, histD_sh.at[pl.ds(s * slot, slot)])
        plsc.subcore_barrier()

        def fire(t, idxS, idxD, semS, semD):
            if t >= T:
                return
            off = (w * T + t) * KD
            pltpu.async_copy(eflat.at[pl.ds(off, KD)], idxS, semS)
            pltpu.async_copy(eflat.at[pl.ds(E + off, KD)], idxD, semD)

        def drain_add(t, idxS, idxD, semS, semD):
            off = (w * T + t) * KD
            pltpu.make_async_copy(eflat.at[pl.ds(off, KD)],
                                  idxS, semS).wait()
            pltpu.make_async_copy(eflat.at[pl.ds(E + off, KD)],
                                  idxD, semD).wait()
            pltpu.sync_copy(ones_v, histS_sh.at[idxS], add=True)
            pltpu.sync_copy(ones_v, histD_sh.at[idxD], add=True)

        IDX = [(idxS0, idxD0, semS0, semD0), (idxS1, idxD1, semS1, semD1)]
        fire(0, *IDX[0])
        fire(1, *IDX[1])
        for t in range(T):
            drain_add(t, *IDX[t % 2])
            fire(t + 2, *IDX[t % 2])
        plsc.subcore_barrier()
        pltpu.sync_copy(histS_sh.at[pl.ds(s * slot, slot)],
                        odeg_hbm.at[pl.ds(c * NPAD + s * slot, slot)])
        pltpu.sync_copy(histD_sh.at[pl.ds(s * slot, slot)],
                        ideg_hbm.at[pl.ds(c * NPAD + s * slot, slot)])

    return deg_kernel


def _make_agg_kernel(E, nc, ns):
    mesh = plsc.VectorSubcoreMesh(core_axis_name="c", subcore_axis_name="s")
    nt = nc * ns
    nblk = E // K
    T = -(-nblk // nt)
    rslot = NPAD // ns
    H = K // 2

    @functools.partial(
        pl.kernel,
        out_type=jax.ShapeDtypeStruct((nc * NPAD, D), jnp.float32),
        mesh=mesh,
        scratch_types=[
            pltpu.VMEM_SHARED((NPAD, D), jnp.float32),
            pltpu.VMEM((K,), jnp.int32),
            pltpu.VMEM((K,), jnp.int32),
            pltpu.VMEM((K,), jnp.int32),
            pltpu.VMEM((K,), jnp.int32),
            pltpu.VMEM((K,), jnp.int32),
            pltpu.VMEM((K,), jnp.int32),
            pltpu.VMEM((K, D), jnp.float32),
            pltpu.SemaphoreType.DMA,
            pltpu.SemaphoreType.DMA,
            pltpu.SemaphoreType.DMA,
            pltpu.SemaphoreType.DMA,
            pltpu.SemaphoreType.DMA,
            pltpu.SemaphoreType.DMA,
            pltpu.SemaphoreType.DMA,
            pltpu.SemaphoreType.DMA,
        ],
    )
    def agg_kernel(fp_hbm, eflat, agg_hbm,
                   agg_sh, idxS0, idxS1, idxS2, idxD0, idxD1, idxD2, rows,
                   isS0, isS1, isS2, isD0, isD1, isD2, gsemA, gsemB):
        c = lax.axis_index("c")
        s = lax.axis_index("s")
        w = c * ns + s
        _zero_vmem_2d(rows, K)
        for t in range(rslot // K):
            pltpu.sync_copy(rows,
                            agg_sh.at[pl.ds(s * rslot + t * K, K)])
        plsc.subcore_barrier()

        def idx_fire(t, idxS, idxD, semS, semD):
            blk = w + nt * t

            @pl.when(blk < nblk)
            def _():
                pltpu.async_copy(eflat.at[pl.ds(blk * K, K)], idxS, semS)
                pltpu.async_copy(eflat.at[pl.ds(E + blk * K, K)], idxD, semD)

        def idx_wait(t, idxS, idxD, semS, semD):
            blk = w + nt * t

            @pl.when(blk < nblk)
            def _():
                pltpu.make_async_copy(eflat.at[pl.ds(blk * K, K)],
                                      idxS, semS).wait()
                pltpu.make_async_copy(eflat.at[pl.ds(E + blk * K, K)],
                                      idxD, semD).wait()

        def gather_scat(t, idxS, idxD):
            blk = w + nt * t

            @pl.when(blk < nblk)
            def _():
                pltpu.async_copy(fp_hbm.at[idxS.at[pl.ds(0, H)]],
                                 rows.at[pl.ds(0, H)], gsemA)
                pltpu.async_copy(fp_hbm.at[idxS.at[pl.ds(H, H)]],
                                 rows.at[pl.ds(H, H)], gsemB)
                pltpu.make_async_copy(fp_hbm.at[idxS.at[pl.ds(0, H)]],
                                      rows.at[pl.ds(0, H)], gsemA).wait()
                pltpu.make_async_copy(fp_hbm.at[idxS.at[pl.ds(H, H)]],
                                      rows.at[pl.ds(H, H)], gsemB).wait()
                pltpu.sync_copy(rows, agg_sh.at[idxD], add=True)

        IDX = [(idxS0, idxD0, isS0, isD0),
               (idxS1, idxD1, isS1, isD1),
               (idxS2, idxD2, isS2, isD2)]
        idx_fire(0, *IDX[0])
        idx_fire(1, *IDX[1])

        def body(i, _):
            for k in range(3):
                t = 3 * i + k
                ibS, ibD, sS, sD = IDX[k]
                idx_wait(t, ibS, ibD, sS, sD)
                ib2 = IDX[(k + 2) % 3]
                idx_fire(t + 2, *ib2)
                gather_scat(t, ibS, ibD)
            return 0

        nloop = -(-T // 3)
        lax.fori_loop(0, nloop, body, 0)
        plsc.subcore_barrier()
        pltpu.sync_copy(agg_sh.at[pl.ds(s * rslot, rslot)],
                        agg_hbm.at[pl.ds(c * NPAD + s * rslot, rslot)])

    return agg_kernel


def _prescale(feats, odp):
    blk = 1000
    grid = N // blk
    nc = odp.shape[0]

    def body(f_ref, d_ref, o_ref):
        deg = d_ref[0]
        for c in range(1, nc):
            deg = deg + d_ref[c]
        norm = lax.rsqrt(jnp.maximum(deg, 1.0))
        o_ref[...] = f_ref[...] * norm

    return pl.pallas_call(
        body,
        grid=(grid,),
        in_specs=[
            pl.BlockSpec((blk, D), lambda i: (i, 0)),
            pl.BlockSpec((nc, blk, 1), lambda i: (0, i, 0)),
        ],
        out_specs=pl.BlockSpec((blk, D), lambda i: (i, 0)),
        out_shape=jax.ShapeDtypeStruct((N, D), jnp.float32),
    )(feats, odp)


def _project(aggp, W, b2, idp, alpha2):
    blk = 1000
    grid = N // blk
    nc = aggp.shape[0]

    def body(a_ref, w_ref, b_ref, d_ref, al_ref, o_ref):
        agg = a_ref[0]
        deg = d_ref[0]
        for c in range(1, nc):
            agg = agg + a_ref[c]
            deg = deg + d_ref[c]
        h = jnp.dot(agg, w_ref[...], preferred_element_type=jnp.float32)
        h = h * lax.rsqrt(jnp.maximum(deg, 1.0)) + b_ref[...]
        a = al_ref[0, 0]
        o_ref[...] = jnp.where(h > 0, h, a * h)

    return pl.pallas_call(
        body,
        grid=(grid,),
        in_specs=[
            pl.BlockSpec((nc, blk, D), lambda i: (0, i, 0)),
            pl.BlockSpec((D, D), lambda i: (0, 0)),
            pl.BlockSpec((1, D), lambda i: (0, 0)),
            pl.BlockSpec((nc, blk, 1), lambda i: (0, i, 0)),
            pl.BlockSpec((1, 1), lambda i: (0, 0)),
        ],
        out_specs=pl.BlockSpec((blk, D), lambda i: (i, 0)),
        out_shape=jax.ShapeDtypeStruct((N, D), jnp.float32),
    )(aggp, W, b2, idp, alpha2)


def kernel(feats, edge_index, W, b, alpha):
    E = edge_index.shape[1]
    assert E % K == 0
    eflat = edge_index.reshape(2 * E)
    info = plsc.get_sparse_core_info()
    nc, ns = info.num_cores, info.num_subcores

    odp, idp = _make_deg_kernel(E, nc, ns)(eflat)
    fp = _prescale(feats, odp.reshape(nc, NPAD, 1))
    aggp = _make_agg_kernel(E, nc, ns)(fp, eflat)
    out = _project(
        aggp.reshape(nc, NPAD, D), W, b.reshape(1, D),
        idp.reshape(nc, NPAD, 1), alpha.reshape(1, 1),
    )
    return out


# speedup vs baseline: 1.2434x; 1.2434x over previous
import functools

import jax
import jax.numpy as jnp
from jax import lax
from jax.experimental import pallas as pl
from jax.experimental.pallas import tpu as pltpu
from jax.experimental.pallas import tpu_sc as plsc

N = 10000
D = 128
NPAD = 10240
K = 128
LANES = 16


def _zero_vmem_1d(ref, n):
    def body(i, _):
        ref[pl.ds(i * LANES, LANES)] = jnp.zeros((LANES,), jnp.float32)
        return 0
    lax.fori_loop(0, n // LANES, body, 0)


def _zero_vmem_2d(ref, rows):
    def body(i, _):
        for k in range(D // LANES):
            ref[i, pl.ds(k * LANES, LANES)] = jnp.zeros((LANES,), jnp.float32)
        return 0
    lax.fori_loop(0, rows, body, 0)


KD = 2000


def _make_deg_kernel(E, nc, ns):
    mesh = plsc.VectorSubcoreMesh(core_axis_name="c", subcore_axis_name="s")
    nt = nc * ns
    T = E // (KD * nt)
    slot = NPAD // ns

    @functools.partial(
        pl.kernel,
        out_type=(
            jax.ShapeDtypeStruct((nc * NPAD,), jnp.float32),
            jax.ShapeDtypeStruct((nc * NPAD,), jnp.float32),
        ),
        mesh=mesh,
        scratch_types=[
            pltpu.VMEM_SHARED((NPAD,), jnp.float32),
            pltpu.VMEM_SHARED((NPAD,), jnp.float32),
            pltpu.VMEM((KD,), jnp.int32),
            pltpu.VMEM((KD,), jnp.int32),
            pltpu.VMEM((KD,), jnp.int32),
            pltpu.VMEM((KD,), jnp.int32),
            pltpu.VMEM((KD,), jnp.float32),
            pltpu.VMEM((slot,), jnp.float32),
            pltpu.SemaphoreType.DMA,
            pltpu.SemaphoreType.DMA,
            pltpu.SemaphoreType.DMA,
            pltpu.SemaphoreType.DMA,
        ],
    )
    def deg_kernel(eflat, odeg_hbm, ideg_hbm,
                   histS_sh, histD_sh,
                   idxS0, idxS1, idxD0, idxD1, ones_v, zbuf,
                   semS0, semS1, semD0, semD1):
        c = lax.axis_index("c")
        s = lax.axis_index("s")
        w = c * ns + s
        _zero_vmem_1d(zbuf, slot)
        for j in range(KD // LANES):
            ones_v[pl.ds(j * LANES, LANES)] = jnp.ones((LANES,), jnp.float32)
        pltpu.sync_copy(zbuf, histS_sh.at[pl.ds(s * slot, slot)])
        pltpu.sync_copy(zbuf, histD_sh.at[pl.ds(s * slot, slot)])
        plsc.subcore_barrier()

        def fire(t, idxS, idxD, semS, semD):
            if t >= T:
                return
            off = (w * T + t) * KD
            pltpu.async_copy(eflat.at[pl.ds(off, KD)], idxS, semS)
            pltpu.async_copy(eflat.at[pl.ds(E + off, KD)], idxD, semD)

        def drain_add(t, idxS, idxD, semS, semD):
            off = (w * T + t) * KD
            pltpu.make_async_copy(eflat.at[pl.ds(off, KD)],
                                  idxS, semS).wait()
            pltpu.make_async_copy(eflat.at[pl.ds(E + off, KD)],
                                  idxD, semD).wait()
            pltpu.sync_copy(ones_v, histS_sh.at[idxS], add=True)
            pltpu.sync_copy(ones_v, histD_sh.at[idxD], add=True)

        IDX = [(idxS0, idxD0, semS0, semD0), (idxS1, idxD1, semS1, semD1)]
        fire(0, *IDX[0])
        fire(1, *IDX[1])
        for t in range(T):
            drain_add(t, *IDX[t % 2])
            fire(t + 2, *IDX[t % 2])
        plsc.subcore_barrier()
        pltpu.sync_copy(histS_sh.at[pl.ds(s * slot, slot)],
                        odeg_hbm.at[pl.ds(c * NPAD + s * slot, slot)])
        pltpu.sync_copy(histD_sh.at[pl.ds(s * slot, slot)],
                        ideg_hbm.at[pl.ds(c * NPAD + s * slot, slot)])

    return deg_kernel


def _make_agg_kernel(E, nc, ns):
    mesh = plsc.VectorSubcoreMesh(core_axis_name="c", subcore_axis_name="s")
    nt = nc * ns
    nblk = E // K
    T = -(-nblk // nt)
    rslot = NPAD // ns

    @functools.partial(
        pl.kernel,
        out_type=jax.ShapeDtypeStruct((nc * NPAD, D), jnp.float32),
        mesh=mesh,
        scratch_types=[
            pltpu.VMEM_SHARED((NPAD, D), jnp.float32),
            pltpu.VMEM((2, K), jnp.int32),
            pltpu.VMEM((2, K), jnp.int32),
            pltpu.VMEM((2, K), jnp.int32),
            pltpu.VMEM((2, K), jnp.int32),
            pltpu.VMEM((K, D), jnp.float32),
            pltpu.VMEM((K, D), jnp.float32),
            pltpu.SemaphoreType.DMA,
            pltpu.SemaphoreType.DMA,
            pltpu.SemaphoreType.DMA,
            pltpu.SemaphoreType.DMA,
            pltpu.SemaphoreType.DMA,
            pltpu.SemaphoreType.DMA,
            pltpu.SemaphoreType.DMA,
            pltpu.SemaphoreType.DMA,
        ],
    )
    def agg_kernel(fp_hbm, eidx3, agg_hbm,
                   agg_sh, idx0, idx1, idx2, idx3, rows0, rows1,
                   isem0, isem1, isem2, isem3,
                   gsem0, gsem1, ssem0, ssem1):
        c = lax.axis_index("c")
        s = lax.axis_index("s")
        w = c * ns + s
        _zero_vmem_2d(rows0, K)
        for t in range(rslot // K):
            pltpu.sync_copy(rows0,
                            agg_sh.at[pl.ds(s * rslot + t * K, K)])
        plsc.subcore_barrier()

        def idx_fire(t, idxb, isem):
            blk = w + nt * t

            @pl.when(blk < nblk)
            def _():
                pltpu.async_copy(eidx3.at[blk], idxb, isem)

        def idx_wait(t, idxb, isem):
            blk = w + nt * t

            @pl.when(blk < nblk)
            def _():
                pltpu.make_async_copy(eidx3.at[blk], idxb, isem).wait()

        H = K // 2

        def g_fire(t, idxb, rb, gsemA, gsemB):
            blk = w + nt * t

            @pl.when(blk < nblk)
            def _():
                pltpu.async_copy(fp_hbm.at[idxb.at[0, pl.ds(0, H)]],
                                 rb.at[pl.ds(0, H)], gsemA)
                pltpu.async_copy(fp_hbm.at[idxb.at[0, pl.ds(H, H)]],
                                 rb.at[pl.ds(H, H)], gsemB)

        def g_wait_scat(t, idxb, rb, gsemA, gsemB):
            blk = w + nt * t

            @pl.when(blk < nblk)
            def _():
                pltpu.make_async_copy(fp_hbm.at[idxb.at[0, pl.ds(0, H)]],
                                      rb.at[pl.ds(0, H)], gsemA).wait()
                pltpu.make_async_copy(fp_hbm.at[idxb.at[0, pl.ds(H, H)]],
                                      rb.at[pl.ds(H, H)], gsemB).wait()
                pltpu.sync_copy(rb, agg_sh.at[idxb.at[1]], add=True)

        IDX = [(idx0, isem0), (idx1, isem1), (idx2, isem2), (idx3, isem3)]
        ROWS = [(rows0, gsem0, ssem0), (rows1, gsem1, ssem1)]
        idx_fire(0, idx0, isem0)
        idx_fire(1, idx1, isem1)
        idx_fire(2, idx2, isem2)
        idx_wait(0, idx0, isem0)
        g_fire(0, idx0, rows0, gsem0, ssem0)

        def body(i, _):
            for k in range(4):
                t = 4 * i + k
                ibt, _ist = IDX[k]
                ib1, is1 = IDX[(k + 1) % 4]
                ib3, is3 = IDX[(k + 3) % 4]
                rbt, gAt, gBt = ROWS[k % 2]
                rb1, gA1, gB1 = ROWS[(k + 1) % 2]
                idx_wait(t + 1, ib1, is1)
                g_fire(t + 1, ib1, rb1, gA1, gB1)
                g_wait_scat(t, ibt, rbt, gAt, gBt)
                idx_fire(t + 3, ib3, is3)
            return 0

        nloop = -(-T // 4)
        lax.fori_loop(0, nloop, body, 0)
        plsc.subcore_barrier()
        pltpu.sync_copy(agg_sh.at[pl.ds(s * rslot, rslot)],
                        agg_hbm.at[pl.ds(c * NPAD + s * rslot, rslot)])

    return agg_kernel


def _prescale(feats, odp):
    blk = 1000
    grid = N // blk
    nc = odp.shape[0]

    def body(f_ref, d_ref, o_ref):
        deg = d_ref[0]
        for c in range(1, nc):
            deg = deg + d_ref[c]
        norm = lax.rsqrt(jnp.maximum(deg, 1.0))
        o_ref[...] = f_ref[...] * norm

    return pl.pallas_call(
        body,
        grid=(grid,),
        in_specs=[
            pl.BlockSpec((blk, D), lambda i: (i, 0)),
            pl.BlockSpec((nc, blk, 1), lambda i: (0, i, 0)),
        ],
        out_specs=pl.BlockSpec((blk, D), lambda i: (i, 0)),
        out_shape=jax.ShapeDtypeStruct((N, D), jnp.float32),
    )(feats, odp)


def _project(aggp, W, b2, idp, alpha2):
    blk = 1000
    grid = N // blk
    nc = aggp.shape[0]

    def body(a_ref, w_ref, b_ref, d_ref, al_ref, o_ref):
        agg = a_ref[0]
        deg = d_ref[0]
        for c in range(1, nc):
            agg = agg + a_ref[c]
            deg = deg + d_ref[c]
        h = jnp.dot(agg, w_ref[...], preferred_element_type=jnp.float32)
        h = h * lax.rsqrt(jnp.maximum(deg, 1.0)) + b_ref[...]
        a = al_ref[0, 0]
        o_ref[...] = jnp.where(h > 0, h, a * h)

    return pl.pallas_call(
        body,
        grid=(grid,),
        in_specs=[
            pl.BlockSpec((nc, blk, D), lambda i: (0, i, 0)),
            pl.BlockSpec((D, D), lambda i: (0, 0)),
            pl.BlockSpec((1, D), lambda i: (0, 0)),
            pl.BlockSpec((nc, blk, 1), lambda i: (0, i, 0)),
            pl.BlockSpec((1, 1), lambda i: (0, 0)),
        ],
        out_specs=pl.BlockSpec((blk, D), lambda i: (i, 0)),
        out_shape=jax.ShapeDtypeStruct((N, D), jnp.float32),
    )(aggp, W, b2, idp, alpha2)


def kernel(feats, edge_index, W, b, alpha):
    E = edge_index.shape[1]
    assert E % K == 0
    eidx3 = jnp.stack(
        [edge_index[0].reshape(E // K, K), edge_index[1].reshape(E // K, K)],
        axis=1)
    info = plsc.get_sparse_core_info()
    nc, ns = info.num_cores, info.num_subcores

    odp, idp = _make_deg_kernel(E, nc, ns)(edge_index.reshape(2 * E))
    fp = _prescale(feats, odp.reshape(nc, NPAD, 1))
    aggp = _make_agg_kernel(E, nc, ns)(fp, eidx3)
    out = _project(
        aggp.reshape(nc, NPAD, D), W, b.reshape(1, D),
        idp.reshape(nc, NPAD, 1), alpha.reshape(1, 1),
    )
    return out
